# Initial kernel scaffold; baseline (speedup 1.0000x reference)
import jax
import jax.numpy as jnp
from jax.experimental import pallas as pl


def kernel(sequence, segment_label, token_table, segment_table):
    raise NotImplementedError("write your pallas kernel here")


# speedup vs baseline: 5.3422x; 5.3422x over previous
import functools
import math

import jax
import jax.numpy as jnp
from jax import lax
from jax.experimental import pallas as pl
from jax.experimental.pallas import tpu as pltpu
from jax.experimental.pallas import tpu_sc as plsc

_LANES = 16


def _combo_tc_body(seg_ref, out_ref):
    R, D = out_ref.shape
    r = lax.broadcasted_iota(jnp.int32, (R, D), 0)
    dcol = lax.broadcasted_iota(jnp.int32, (R, D), 1)
    l3 = r // 3
    s = r - 3 * l3
    half = (dcol // 2).astype(jnp.float32)
    div = jnp.exp(half * (-2.0 * math.log(10000.0) / D))
    ang = l3.astype(jnp.float32) * div
    pe = jnp.where(dcol % 2 == 0, jnp.sin(ang), jnp.cos(ang))
    st = seg_ref[...]
    seg0 = jnp.broadcast_to(st[0:1, :], (R, D))
    seg1 = jnp.broadcast_to(st[1:2, :], (R, D))
    seg2 = jnp.broadcast_to(st[2:3, :], (R, D))
    out_ref[...] = pe + jnp.where(s == 0, seg0, jnp.where(s == 1, seg1, seg2))


def _build_combo(segment_table, rows):
    return pl.pallas_call(
        _combo_tc_body,
        out_shape=jax.ShapeDtypeStruct((rows, segment_table.shape[1]), jnp.float32),
    )(segment_table)


def _sc_lookup(seq_flat, lbl_flat, token_table, combo, L):
    N = seq_flat.shape[0]
    D = token_table.shape[1]
    info = plsc.get_sparse_core_info()
    NC, NS = info.num_cores, info.num_subcores
    NW = NC * NS
    C = 128
    assert N % (NW * C) == 0 and D % _LANES == 0
    rows_per_w = N // NW
    chunks = rows_per_w // C
    assert rows_per_w % L == 0 and C <= L
    mesh = plsc.VectorSubcoreMesh(core_axis_name="c", subcore_axis_name="s")

    @functools.partial(
        pl.kernel,
        out_type=jax.ShapeDtypeStruct((N, D), jnp.float32),
        mesh=mesh,
        scratch_types=[
            pltpu.VMEM((C,), jnp.int32),
            pltpu.VMEM((C,), jnp.int32),
            pltpu.VMEM((C, D), jnp.float32),
            pltpu.VMEM((C, D), jnp.float32),
            pltpu.SemaphoreType.DMA,
            pltpu.SemaphoreType.DMA,
        ],
    )
    def k(seq_hbm, lbl_hbm, tok_hbm, combo_hbm, out_hbm,
          sidx_v, cidx_v, tok_v, cmb_v, sem_t, sem_c):
        wid = lax.axis_index("s") * NC + lax.axis_index("c")
        tile_base = wid * rows_per_w

        def chunk(c, lpos0):
            base = tile_base + c * C
            pltpu.sync_copy(seq_hbm.at[pl.ds(base, C)], sidx_v)
            pltpu.sync_copy(lbl_hbm.at[pl.ds(base, C)], cidx_v)
            tok_dma = pltpu.async_copy(tok_hbm.at[sidx_v], tok_v, sem_t)
            for j in range(C // _LANES):
                v = lpos0 + (j * _LANES + lax.iota(jnp.int32, _LANES))
                lpos = jnp.where(v >= L, v - L, v)
                sl = pl.ds(j * _LANES, _LANES)
                cidx_v[sl] = 3 * lpos + cidx_v[sl]
            cmb_dma = pltpu.async_copy(combo_hbm.at[cidx_v], cmb_v, sem_c)
            tok_dma.wait()
            cmb_dma.wait()

            def addrow(r, carry2):
                for u in range(D // _LANES):
                    sl = pl.ds(u * _LANES, _LANES)
                    tok_v[r, sl] = tok_v[r, sl] + cmb_v[r, sl]
                return carry2

            lax.fori_loop(0, C, addrow, 0, unroll=False)
            pltpu.sync_copy(tok_v, out_hbm.at[pl.ds(base, C)])
            nxt = lpos0 + C
            return jnp.where(nxt >= L, nxt - L, nxt)

        lax.fori_loop(0, chunks, chunk, jnp.int32(0), unroll=False)

    return k(seq_flat, lbl_flat, token_table, combo)


def kernel(sequence, segment_label, token_table, segment_table):
    B, L = sequence.shape
    D = token_table.shape[1]
    combo_rows = ((3 * L + 7) // 8) * 8
    combo = _build_combo(segment_table, combo_rows)
    seq_flat = sequence.reshape(-1).astype(jnp.int32)
    lbl_flat = segment_label.reshape(-1).astype(jnp.int32)
    out = _sc_lookup(seq_flat, lbl_flat, token_table, combo, L)
    return out.reshape(B, L, D)
